# baseline (speedup 1.0000x reference)
import functools

import jax
import jax.numpy as jnp
from jax import lax
from jax.experimental import pallas as pl
from jax.experimental.pallas import tpu as pltpu
from jax.experimental.pallas import tpu_sc as plsc

B = 16384
D = 128
NC = 2
NS = 16
NW = NC * NS
ROWS_PER_W = B // NW
CHUNK = 128
NCHUNK = ROWS_PER_W // CHUNK
LANES = 16
GROUPS = D // LANES
RUNROLL = 4


def _sc_partial_sums(features, labels, centers):
    mesh = plsc.VectorSubcoreMesh(core_axis_name="c", subcore_axis_name="s")

    @functools.partial(
        pl.kernel,
        out_type=jax.ShapeDtypeStruct((NW * LANES,), jnp.float32),
        mesh=mesh,
        scratch_types=[
            pltpu.VMEM((ROWS_PER_W,), jnp.int32),
            pltpu.VMEM((2, CHUNK, D), jnp.float32),
            pltpu.VMEM((2, CHUNK, D), jnp.float32),
            pltpu.VMEM((LANES,), jnp.float32),
            pltpu.SemaphoreType.DMA,
            pltpu.SemaphoreType.DMA,
        ],
    )
    def body(feat_hbm, lab_hbm, cent_hbm, out_hbm, idx_v, cent_v, feat_v,
             acc_v, sem0, sem1):
        cid = lax.axis_index("c")
        sid = lax.axis_index("s")
        wid = sid * NC + cid
        base = wid * ROWS_PER_W
        pltpu.sync_copy(lab_hbm.at[pl.ds(base, ROWS_PER_W)], idx_v)
        sems = (sem0, sem1)

        def start(j):
            s = sems[j % 2]
            g = pltpu.async_copy(
                cent_hbm.at[idx_v.at[pl.ds(j * CHUNK, CHUNK)]],
                cent_v.at[j % 2], s)
            f = pltpu.async_copy(feat_hbm.at[pl.ds(base + j * CHUNK, CHUNK)],
                                 feat_v.at[j % 2], s)
            return (g, f)

        copies = [None] * NCHUNK
        copies[0] = start(0)
        accs = tuple(jnp.zeros((LANES,), jnp.float32) for _ in range(GROUPS))
        for j in range(NCHUNK):
            if j + 1 < NCHUNK:
                copies[j + 1] = start(j + 1)
            gcp, fcp = copies[j]
            gcp.wait()
            fcp.wait()
            b = j % 2

            def row_body(r, accs):
                out = list(accs)
                for u in range(RUNROLL):
                    for g in range(GROUPS):
                        f = feat_v[b, r * RUNROLL + u, pl.ds(g * LANES, LANES)]
                        c = cent_v[b, r * RUNROLL + u, pl.ds(g * LANES, LANES)]
                        d = f - c
                        out[g] = out[g] + d * d
                return tuple(out)

            accs = lax.fori_loop(0, CHUNK // RUNROLL, row_body, accs)
        acc = accs[0]
        for g in range(1, GROUPS):
            acc = acc + accs[g]
        acc_v[...] = acc
        pltpu.sync_copy(acc_v, out_hbm.at[pl.ds(wid * LANES, LANES)])

    return body(features, labels, centers)


def kernel(features, labels, centers):
    labels = labels.astype(jnp.int32)
    partials = _sc_partial_sums(features, labels, centers)
    return jnp.sum(partials) / (2.0 * features.shape[0])


# speedup vs baseline: 1.0401x; 1.0401x over previous
import functools

import jax
import jax.numpy as jnp
from jax import lax
from jax.experimental import pallas as pl
from jax.experimental.pallas import tpu as pltpu
from jax.experimental.pallas import tpu_sc as plsc

B = 16384
D = 128
NC = 2
NS = 16
NW = NC * NS
ROWS_PER_W = B // NW
CHUNK = 128
NCHUNK = ROWS_PER_W // CHUNK
LANES = 16
GROUPS = D // LANES
RUNROLL = 1
LUNROLL = 4


def _sc_partial_sums(features, labels, centers):
    mesh = plsc.VectorSubcoreMesh(core_axis_name="c", subcore_axis_name="s")

    @functools.partial(
        pl.kernel,
        out_type=jax.ShapeDtypeStruct((NW * LANES,), jnp.float32),
        mesh=mesh,
        scratch_types=[
            pltpu.VMEM((ROWS_PER_W,), jnp.int32),
            pltpu.VMEM((2, CHUNK, D), jnp.float32),
            pltpu.VMEM((2, CHUNK, D), jnp.float32),
            pltpu.VMEM((LANES,), jnp.float32),
            pltpu.SemaphoreType.DMA,
            pltpu.SemaphoreType.DMA,
        ],
    )
    def body(feat_hbm, lab_hbm, cent_hbm, out_hbm, idx_v, cent_v, feat_v,
             acc_v, sem0, sem1):
        cid = lax.axis_index("c")
        sid = lax.axis_index("s")
        wid = sid * NC + cid
        base = wid * ROWS_PER_W
        pltpu.sync_copy(lab_hbm.at[pl.ds(base, ROWS_PER_W)], idx_v)
        sems = (sem0, sem1)

        def start(j):
            s = sems[j % 2]
            g = pltpu.async_copy(
                cent_hbm.at[idx_v.at[pl.ds(j * CHUNK, CHUNK)]],
                cent_v.at[j % 2], s)
            f = pltpu.async_copy(feat_hbm.at[pl.ds(base + j * CHUNK, CHUNK)],
                                 feat_v.at[j % 2], s)
            return (g, f)

        copies = [None] * NCHUNK
        copies[0] = start(0)
        accs = tuple(jnp.zeros((LANES,), jnp.float32) for _ in range(GROUPS))
        for j in range(NCHUNK):
            if j + 1 < NCHUNK:
                copies[j + 1] = start(j + 1)
            gcp, fcp = copies[j]
            gcp.wait()
            fcp.wait()
            b = j % 2

            @plsc.parallel_loop(0, CHUNK, 1, unroll=LUNROLL, carry=accs)
            def accs(r, accs):
                out = list(accs)
                for g in range(GROUPS):
                    f = feat_v[b, r, pl.ds(g * LANES, LANES)]
                    c = cent_v[b, r, pl.ds(g * LANES, LANES)]
                    d = f - c
                    out[g] = out[g] + d * d
                return tuple(out)
        acc = accs[0]
        for g in range(1, GROUPS):
            acc = acc + accs[g]
        acc_v[...] = acc
        pltpu.sync_copy(acc_v, out_hbm.at[pl.ds(wid * LANES, LANES)])

    return body(features, labels, centers)


def kernel(features, labels, centers):
    labels = labels.astype(jnp.int32)
    partials = _sc_partial_sums(features, labels, centers)
    return jnp.sum(partials) / (2.0 * features.shape[0])


# speedup vs baseline: 1.0412x; 1.0010x over previous
import functools

import jax
import jax.numpy as jnp
from jax import lax
from jax.experimental import pallas as pl
from jax.experimental.pallas import tpu as pltpu
from jax.experimental.pallas import tpu_sc as plsc

B = 16384
D = 128
NC = 2
NS = 16
NW = NC * NS
ROWS_PER_W = B // NW
CHUNK = 128
NCHUNK = ROWS_PER_W // CHUNK
LANES = 16
GROUPS = D // LANES
RUNROLL = 1
LUNROLL = 8


def _sc_partial_sums(features, labels, centers):
    mesh = plsc.VectorSubcoreMesh(core_axis_name="c", subcore_axis_name="s")

    @functools.partial(
        pl.kernel,
        out_type=jax.ShapeDtypeStruct((NW * LANES,), jnp.float32),
        mesh=mesh,
        scratch_types=[
            pltpu.VMEM((ROWS_PER_W,), jnp.int32),
            pltpu.VMEM((2, CHUNK, D), jnp.float32),
            pltpu.VMEM((2, CHUNK, D), jnp.float32),
            pltpu.VMEM((LANES,), jnp.float32),
            pltpu.SemaphoreType.DMA,
            pltpu.SemaphoreType.DMA,
        ],
    )
    def body(feat_hbm, lab_hbm, cent_hbm, out_hbm, idx_v, cent_v, feat_v,
             acc_v, sem0, sem1):
        cid = lax.axis_index("c")
        sid = lax.axis_index("s")
        wid = sid * NC + cid
        base = wid * ROWS_PER_W
        pltpu.sync_copy(lab_hbm.at[pl.ds(base, ROWS_PER_W)], idx_v)
        sems = (sem0, sem1)

        def start(j):
            s = sems[j % 2]
            g = pltpu.async_copy(
                cent_hbm.at[idx_v.at[pl.ds(j * CHUNK, CHUNK)]],
                cent_v.at[j % 2], s)
            f = pltpu.async_copy(feat_hbm.at[pl.ds(base + j * CHUNK, CHUNK)],
                                 feat_v.at[j % 2], s)
            return (g, f)

        copies = [None] * NCHUNK
        copies[0] = start(0)
        accs = tuple(jnp.zeros((LANES,), jnp.float32) for _ in range(GROUPS))
        for j in range(NCHUNK):
            if j + 1 < NCHUNK:
                copies[j + 1] = start(j + 1)
            gcp, fcp = copies[j]
            gcp.wait()
            fcp.wait()
            b = j % 2

            @plsc.parallel_loop(0, CHUNK, 1, unroll=LUNROLL, carry=accs)
            def accs(r, accs):
                out = list(accs)
                for g in range(GROUPS):
                    f = feat_v[b, r, pl.ds(g * LANES, LANES)]
                    c = cent_v[b, r, pl.ds(g * LANES, LANES)]
                    d = f - c
                    out[g] = out[g] + d * d
                return tuple(out)
        acc = accs[0]
        for g in range(1, GROUPS):
            acc = acc + accs[g]
        acc_v[...] = acc
        pltpu.sync_copy(acc_v, out_hbm.at[pl.ds(wid * LANES, LANES)])

    return body(features, labels, centers)


def kernel(features, labels, centers):
    labels = labels.astype(jnp.int32)
    partials = _sc_partial_sums(features, labels, centers)
    return jnp.sum(partials) / (2.0 * features.shape[0])


# speedup vs baseline: 1.0440x; 1.0027x over previous
import functools

import jax
import jax.numpy as jnp
from jax import lax
from jax.experimental import pallas as pl
from jax.experimental.pallas import tpu as pltpu
from jax.experimental.pallas import tpu_sc as plsc

B = 16384
D = 128
NC = 2
NS = 16
NW = NC * NS
ROWS_PER_W = B // NW
CHUNK = 128
NCHUNK = ROWS_PER_W // CHUNK
LANES = 16
GROUPS = D // LANES
RUNROLL = 1
LUNROLL = 4


def _sc_partial_sums(features, labels, centers):
    mesh = plsc.VectorSubcoreMesh(core_axis_name="c", subcore_axis_name="s")

    @functools.partial(
        pl.kernel,
        out_type=jax.ShapeDtypeStruct((NW * LANES,), jnp.float32),
        mesh=mesh,
        scratch_types=[
            pltpu.VMEM((ROWS_PER_W,), jnp.int32),
            pltpu.VMEM((2, CHUNK, D), jnp.float32),
            pltpu.VMEM((2, CHUNK, D), jnp.float32),
            pltpu.VMEM((LANES,), jnp.float32),
            pltpu.SemaphoreType.DMA,
            pltpu.SemaphoreType.DMA,
        ],
    )
    def body(feat_hbm, lab_hbm, cent_hbm, out_hbm, idx_v, cent_v, feat_v,
             acc_v, sem0, sem1):
        cid = lax.axis_index("c")
        sid = lax.axis_index("s")
        wid = sid * NC + cid
        base = wid * ROWS_PER_W
        pltpu.sync_copy(lab_hbm.at[pl.ds(base, ROWS_PER_W)], idx_v)
        sems = (sem0, sem1)

        def start(j):
            s = sems[j % 2]
            g = pltpu.async_copy(
                cent_hbm.at[idx_v.at[pl.ds(j * CHUNK, CHUNK)]],
                cent_v.at[j % 2], s)
            f = pltpu.async_copy(feat_hbm.at[pl.ds(base + j * CHUNK, CHUNK)],
                                 feat_v.at[j % 2], s)
            return (g, f)

        copies = [None] * NCHUNK
        copies[0] = start(0)
        accs = tuple(jnp.zeros((LANES,), jnp.float32) for _ in range(GROUPS))
        for j in range(NCHUNK):
            if j + 1 < NCHUNK:
                copies[j + 1] = start(j + 1)
            gcp, fcp = copies[j]
            gcp.wait()
            fcp.wait()
            b = j % 2

            @plsc.parallel_loop(0, CHUNK, 1, unroll=LUNROLL, carry=accs)
            def accs(r, accs):
                out = list(accs)
                for g in range(GROUPS):
                    f = feat_v[b, r, pl.ds(g * LANES, LANES)]
                    c = cent_v[b, r, pl.ds(g * LANES, LANES)]
                    d = f - c
                    out[g] = out[g] + d * d
                return tuple(out)
        acc = accs[0]
        for g in range(1, GROUPS):
            acc = acc + accs[g]
        acc_v[...] = acc
        pltpu.sync_copy(acc_v, out_hbm.at[pl.ds(wid * LANES, LANES)])

    return body(features, labels, centers)


def kernel(features, labels, centers):
    labels = labels.astype(jnp.int32)
    partials = _sc_partial_sums(features, labels, centers)
    return jnp.sum(partials) / (2.0 * features.shape[0])
